# baseline (speedup 1.0000x reference)
import functools

import jax
import jax.numpy as jnp
from jax import lax
from jax.experimental import pallas as pl
from jax.experimental.pallas import tpu as pltpu
from jax.experimental.pallas import tpu_sc as plsc

VOCAB = 100000
HIDDEN = 128
B = 1024
L = 50
NLANE = 16
NGRP = HIDDEN // NLANE

NC = 2
NS = 16
NW = NC * NS

TOTAL = B * L
PER_W = TOTAL // NW
SEQS = B // NW
P_CHUNK = 5
N_CHUNKS = L // P_CHUNK
CHUNK = P_CHUNK * SEQS
NBUF = 4


def _sc_embed(ids_perm, token_embedding, pos_block):
    mesh = plsc.VectorSubcoreMesh(core_axis_name="c", subcore_axis_name="s")

    @functools.partial(
        pl.kernel,
        out_type=jax.ShapeDtypeStruct((L, B, HIDDEN), jnp.float32),
        mesh=mesh,
        scratch_types=[
            pltpu.VMEM((PER_W,), jnp.int32),
            pltpu.VMEM((L, HIDDEN), jnp.float32),
            pltpu.VMEM((CHUNK, HIDDEN), jnp.float32),
            pltpu.VMEM((CHUNK, HIDDEN), jnp.float32),
            pltpu.VMEM((CHUNK, HIDDEN), jnp.float32),
            pltpu.VMEM((CHUNK, HIDDEN), jnp.float32),
            pltpu.SemaphoreType.DMA,
            pltpu.SemaphoreType.DMA,
            pltpu.SemaphoreType.DMA,
            pltpu.SemaphoreType.DMA,
            pltpu.SemaphoreType.DMA,
            pltpu.SemaphoreType.DMA,
            pltpu.SemaphoreType.DMA,
            pltpu.SemaphoreType.DMA,
            pltpu.SemaphoreType.DMA,
        ],
    )
    def k(ids_hbm, table_hbm, pos_hbm, out_hbm,
          idx_v, pos_v, rows0, rows1, rows2, rows3,
          gsem0, gsem1, gsem2, gsem3, osem0, osem1, osem2, osem3, psem):
        wid = lax.axis_index("s") * NC + lax.axis_index("c")
        base = wid * PER_W
        pcp = pltpu.async_copy(pos_hbm, pos_v, psem)
        pltpu.sync_copy(ids_hbm.at[pl.ds(base, PER_W)], idx_v)

        rows = (rows0, rows1, rows2, rows3)
        gsems = (gsem0, gsem1, gsem2, gsem3)
        osems = (osem0, osem1, osem2, osem3)
        seq_base = wid * SEQS

        def start_gather(g):
            return pltpu.async_copy(
                table_hbm.at[idx_v.at[pl.ds(g * CHUNK, CHUNK)]],
                rows[g % NBUF], gsems[g % NBUF])

        def add_pos(g):
            rv = rows[g % NBUF]

            def body(i, carry):
                p = g * P_CHUNK + i
                regs = [pos_v[p, pl.ds(c * NLANE, NLANE)] for c in range(NGRP)]
                row0 = i * SEQS
                for s in range(SEQS):
                    for c in range(NGRP):
                        plsc.addupdate(
                            rv.at[row0 + s, pl.ds(c * NLANE, NLANE)], regs[c])
                return carry

            lax.fori_loop(0, P_CHUNK, body, 0, unroll=False)

        def start_out(g):
            rv = rows[g % NBUF]
            return [pltpu.async_copy(
                        rv.at[pl.ds(i * SEQS, SEQS)],
                        out_hbm.at[g * P_CHUNK + i, pl.ds(seq_base, SEQS)],
                        osems[g % NBUF])
                    for i in range(P_CHUNK)]

        gcp = [None] * N_CHUNKS
        ocp = [None] * N_CHUNKS
        for g in range(NBUF - 1):
            gcp[g] = start_gather(g)
        pcp.wait()
        for g in range(N_CHUNKS):
            gcp[g].wait()
            add_pos(g)
            ocp[g] = start_out(g)
            if g + NBUF - 1 < N_CHUNKS:
                if g >= 1:
                    for cp in ocp[g - 1]:
                        cp.wait()
                gcp[g + NBUF - 1] = start_gather(g + NBUF - 1)
        for g in range(N_CHUNKS - NBUF, N_CHUNKS):
            if g >= 0:
                for cp in ocp[g]:
                    cp.wait()

    return k(ids_perm, token_embedding, pos_block)


def kernel(input_ids, token_embedding, position_embedding):
    ids_perm = (input_ids.astype(jnp.int32)
                .reshape(NW, SEQS, N_CHUNKS, P_CHUNK)
                .transpose(0, 2, 3, 1)
                .reshape(TOTAL))
    pos_block = position_embedding[0, :L, :]
    out_t = _sc_embed(ids_perm, token_embedding, pos_block)
    return jnp.transpose(out_t, (1, 0, 2))


# speedup vs baseline: 1.0667x; 1.0667x over previous
import functools

import jax
import jax.numpy as jnp
from jax import lax
from jax.experimental import pallas as pl
from jax.experimental.pallas import tpu as pltpu
from jax.experimental.pallas import tpu_sc as plsc

VOCAB = 100000
HIDDEN = 128
MAX_POS = 200
B = 1024
L = 50
NLANE = 16
NGRP = HIDDEN // NLANE

NC = 2
NS = 16
NW = NC * NS

TOTAL = B * L
PER_W = TOTAL // NW
SEQS = B // NW
P_CHUNK = 10
N_CHUNKS = L // P_CHUNK
CHUNK = P_CHUNK * SEQS


def _sc_embed(input_ids, token_embedding, pos_table):
    mesh = plsc.VectorSubcoreMesh(core_axis_name="c", subcore_axis_name="s")

    @functools.partial(
        pl.kernel,
        out_type=jax.ShapeDtypeStruct((L, B, HIDDEN), jnp.float32),
        mesh=mesh,
        scratch_types=[
            pltpu.VMEM((PER_W,), jnp.int32),
            pltpu.VMEM((56, HIDDEN), jnp.float32),
            pltpu.VMEM((CHUNK, HIDDEN), jnp.float32),
            pltpu.VMEM((CHUNK, HIDDEN), jnp.float32),
            pltpu.SemaphoreType.DMA,
            pltpu.SemaphoreType.DMA,
            pltpu.SemaphoreType.DMA,
            pltpu.SemaphoreType.DMA,
            pltpu.SemaphoreType.DMA,
        ],
    )
    def k(ids_hbm, table_hbm, pos_hbm, out_hbm,
          idx_v, pos_v, rows0, rows1,
          gsem0, gsem1, osem0, osem1, psem):
        wid = lax.axis_index("s") * NC + lax.axis_index("c")
        seq_base = wid * SEQS
        pcp = pltpu.async_copy(pos_hbm.at[pl.ds(0, 56)], pos_v, psem)
        pltpu.sync_copy(ids_hbm.at[pl.ds(wid * PER_W, PER_W)], idx_v)

        rows = (rows0, rows1)
        gsems = (gsem0, gsem1)
        osems = (osem0, osem1)

        def start_gather(g):
            return pltpu.async_copy(
                table_hbm.at[idx_v.at[pl.ds(g * CHUNK, CHUNK)]],
                rows[g % 2], gsems[g % 2])

        def add_pos(g):
            rv = rows[g % 2]

            def body(i, carry):
                p = g * P_CHUNK + i
                regs = [pos_v[p, pl.ds(c * NLANE, NLANE)] for c in range(NGRP)]
                row0 = i * SEQS
                for s in range(SEQS):
                    for c in range(NGRP):
                        plsc.addupdate(
                            rv.at[row0 + s, pl.ds(c * NLANE, NLANE)], regs[c])
                return carry

            lax.fori_loop(0, P_CHUNK, body, 0, unroll=False)

        def start_out(g):
            rv = rows[g % 2]
            return [pltpu.async_copy(
                        rv.at[pl.ds(i * SEQS, SEQS)],
                        out_hbm.at[g * P_CHUNK + i, pl.ds(seq_base, SEQS)],
                        osems[g % 2])
                    for i in range(P_CHUNK)]

        gcp = [None] * N_CHUNKS
        ocp = [None] * N_CHUNKS
        gcp[0] = start_gather(0)
        pcp.wait()
        for g in range(N_CHUNKS):
            if g + 1 < N_CHUNKS:
                if g >= 1:
                    for cp in ocp[g - 1]:
                        cp.wait()
                gcp[g + 1] = start_gather(g + 1)
            gcp[g].wait()
            add_pos(g)
            ocp[g] = start_out(g)
        for cp in ocp[N_CHUNKS - 2]:
            cp.wait()
        for cp in ocp[N_CHUNKS - 1]:
            cp.wait()

    return k(input_ids, token_embedding, pos_table)


def kernel(input_ids, token_embedding, position_embedding):
    ids_perm = (input_ids.astype(jnp.int32)
                .reshape(NW, SEQS, N_CHUNKS, P_CHUNK)
                .transpose(0, 2, 3, 1)
                .reshape(TOTAL))
    pos_table = position_embedding.reshape(MAX_POS, HIDDEN)
    out_t = _sc_embed(ids_perm, token_embedding, pos_table)
    return jnp.transpose(out_t, (1, 0, 2))


# speedup vs baseline: 1.0772x; 1.0098x over previous
import functools

import jax
import jax.numpy as jnp
from jax import lax
from jax.experimental import pallas as pl
from jax.experimental.pallas import tpu as pltpu
from jax.experimental.pallas import tpu_sc as plsc

VOCAB = 100000
HIDDEN = 128
MAX_POS = 200
B = 1024
L = 50
NLANE = 16
NGRP = HIDDEN // NLANE

NC = 2
NS = 16
NW = NC * NS

TOTAL = B * L
PER_W = TOTAL // NW
SEQS = B // NW
P_CHUNK = 10
N_CHUNKS = L // P_CHUNK
CHUNK = P_CHUNK * SEQS


def _sc_embed(ids_perm, token_embedding, pos_block):
    mesh = plsc.VectorSubcoreMesh(core_axis_name="c", subcore_axis_name="s")

    @functools.partial(
        pl.kernel,
        out_type=jax.ShapeDtypeStruct((L, B, HIDDEN), jnp.float32),
        mesh=mesh,
        scratch_types=[
            pltpu.VMEM((PER_W,), jnp.int32),
            pltpu.VMEM((L, HIDDEN), jnp.float32),
            pltpu.VMEM((CHUNK, HIDDEN), jnp.float32),
            pltpu.VMEM((CHUNK, HIDDEN), jnp.float32),
            pltpu.SemaphoreType.DMA,
            pltpu.SemaphoreType.DMA,
            pltpu.SemaphoreType.DMA,
            pltpu.SemaphoreType.DMA,
            pltpu.SemaphoreType.DMA,
        ],
    )
    def k(ids_hbm, table_hbm, pos_hbm, out_hbm,
          idx_v, pos_v, rows0, rows1,
          gsem0, gsem1, osem0, osem1, psem):
        wid = lax.axis_index("s") * NC + lax.axis_index("c")
        seq_base = wid * SEQS
        pcp = pltpu.async_copy(pos_hbm, pos_v, psem)
        pltpu.sync_copy(ids_hbm.at[pl.ds(wid * PER_W, PER_W)], idx_v)

        rows = (rows0, rows1)
        gsems = (gsem0, gsem1)
        osems = (osem0, osem1)

        def start_gather(g):
            return pltpu.async_copy(
                table_hbm.at[idx_v.at[pl.ds(g * CHUNK, CHUNK)]],
                rows[g % 2], gsems[g % 2])

        def add_pos(g):
            rv = rows[g % 2]

            def body(i, carry):
                p = g * P_CHUNK + i
                regs = [pos_v[p, pl.ds(c * NLANE, NLANE)] for c in range(NGRP)]
                row0 = i * SEQS
                for s in range(SEQS):
                    for c in range(NGRP):
                        plsc.addupdate(
                            rv.at[row0 + s, pl.ds(c * NLANE, NLANE)], regs[c])
                return carry

            lax.fori_loop(0, P_CHUNK, body, 0, unroll=False)

        def start_out(g):
            rv = rows[g % 2]
            return [pltpu.async_copy(
                        rv.at[pl.ds(i * SEQS, SEQS)],
                        out_hbm.at[g * P_CHUNK + i, pl.ds(seq_base, SEQS)],
                        osems[g % 2])
                    for i in range(P_CHUNK)]

        gcp = [None] * N_CHUNKS
        ocp = [None] * N_CHUNKS
        gcp[0] = start_gather(0)
        pcp.wait()
        for g in range(N_CHUNKS):
            if g + 1 < N_CHUNKS:
                if g >= 1:
                    for cp in ocp[g - 1]:
                        cp.wait()
                gcp[g + 1] = start_gather(g + 1)
            gcp[g].wait()
            add_pos(g)
            ocp[g] = start_out(g)
        for cp in ocp[N_CHUNKS - 2]:
            cp.wait()
        for cp in ocp[N_CHUNKS - 1]:
            cp.wait()

    return k(ids_perm, token_embedding, pos_block)


def kernel(input_ids, token_embedding, position_embedding):
    ids_perm = (input_ids.astype(jnp.int32)
                .reshape(NW, SEQS, N_CHUNKS, P_CHUNK)
                .transpose(0, 2, 3, 1)
                .reshape(TOTAL))
    pos_block = position_embedding[0, :L, :]
    out_t = _sc_embed(ids_perm, token_embedding, pos_block)
    return jnp.transpose(out_t, (1, 0, 2))
